# baseline (speedup 1.0000x reference)
import functools

import jax
import jax.numpy as jnp
from jax import lax
from jax.experimental import pallas as pl
from jax.experimental.pallas import tpu as pltpu
from jax.experimental.pallas import tpu_sc as plsc

_LANES = 16
_CHUNK = 4


@functools.lru_cache(maxsize=None)
def _make_gather(B, D):
    info = plsc.get_sparse_core_info()
    NC, NS = info.num_cores, info.num_subcores
    NW = NC * NS
    b_per_w = B // NW
    n_chunks = b_per_w // _CHUNK
    assert b_per_w * NW == B and n_chunks * _CHUNK == b_per_w

    mesh = plsc.VectorSubcoreMesh(core_axis_name="c", subcore_axis_name="s")

    @functools.partial(
        pl.kernel,
        mesh=mesh,
        out_type=[
            jax.ShapeDtypeStruct((B, D), jnp.float32),
            jax.ShapeDtypeStruct((B, D), jnp.float32),
        ],
        scratch_types=[
            pltpu.VMEM((b_per_w,), jnp.int32),
            pltpu.VMEM((_CHUNK, D, 128), jnp.float32),
            pltpu.VMEM((_CHUNK, D, 128), jnp.float32),
            pltpu.VMEM((2 * _CHUNK, D), jnp.float32),
            pltpu.SemaphoreType.DMA,
            pltpu.SemaphoreType.DMA,
        ],
        compiler_params=pltpu.CompilerParams(needs_layout_passes=False),
    )
    def gather_kernel(uids, iids, utabT, itabT, uout, iout,
                      ids_v, blk_a, blk_b, rows_v, sem_a, sem_b):
        wid = lax.axis_index("s") * NC + lax.axis_index("c")
        base = wid * b_per_w
        iota = lax.iota(jnp.int32, _LANES)

        for ids_hbm, tabT, out in ((uids, utabT, uout), (iids, itabT, iout)):
            pltpu.sync_copy(ids_hbm.at[pl.ds(base, b_per_w)], ids_v)

            def fire(c, blk, sem):
                idv = ids_v[pl.ds(c * _CHUNK, _LANES)]
                for q in range(_CHUNK):
                    col0 = pl.multiple_of((idv[q] >> 7) << 7, 128)
                    for a in range(D // 8):
                        pltpu.async_copy(
                            tabT.at[pl.ds(a * 8, 8), pl.ds(col0, 128)],
                            blk.at[q, pl.ds(a * 8, 8)], sem)

            def drain_select(c, blk, sem, half):
                pltpu.make_async_copy(
                    tabT.at[:, pl.ds(0, 128)], blk, sem).wait()
                idv = ids_v[pl.ds(c * _CHUNK, _LANES)]
                for q in range(_CHUNK):
                    lane = jnp.full((_LANES,), idv[q] & 127, jnp.int32)
                    slab = jnp.full((_LANES,), q, jnp.int32)
                    for k in range(D // _LANES):
                        vals = plsc.load_gather(
                            blk, [slab, iota + k * _LANES, lane])
                        plsc.store_scatter(
                            rows_v,
                            [jnp.full((_LANES,), half * _CHUNK + q,
                                      jnp.int32),
                             iota + k * _LANES],
                            vals)

            def pair_body(p, carry):
                c0 = 2 * p
                fire(c0 + 1, blk_b, sem_b)
                drain_select(c0, blk_a, sem_a, 0)

                @pl.when(c0 + 2 < n_chunks)
                def _():
                    fire(c0 + 2, blk_a, sem_a)

                drain_select(c0 + 1, blk_b, sem_b, 1)
                pltpu.sync_copy(
                    rows_v, out.at[pl.ds(base + c0 * _CHUNK, 2 * _CHUNK)])
                return carry

            fire(0, blk_a, sem_a)
            lax.fori_loop(0, n_chunks // 2, pair_body, 0)

    return gather_kernel


def _mlp_body(eu, ei, w1u, w1i, b1, w2, b2, wp, bp, out):
    x = jnp.dot(eu[...], w1u[...], preferred_element_type=jnp.float32)
    x = x + jnp.dot(ei[...], w1i[...], preferred_element_type=jnp.float32)
    h = jnp.maximum(x + b1[...], 0.0)
    h = jnp.maximum(
        jnp.dot(h, w2[...], preferred_element_type=jnp.float32) + b2[...], 0.0)
    out[...] = jnp.sum(h * wp[...], axis=1, keepdims=True) + bp[...]


@functools.lru_cache(maxsize=None)
def _make_mlp(B, D, H1, H2, BM):
    grid = (B // BM,)
    return pl.pallas_call(
        _mlp_body,
        grid=grid,
        in_specs=[
            pl.BlockSpec((BM, D), lambda i: (i, 0)),
            pl.BlockSpec((BM, D), lambda i: (i, 0)),
            pl.BlockSpec((D, H1), lambda i: (0, 0)),
            pl.BlockSpec((D, H1), lambda i: (0, 0)),
            pl.BlockSpec((1, H1), lambda i: (0, 0)),
            pl.BlockSpec((H1, H2), lambda i: (0, 0)),
            pl.BlockSpec((1, H2), lambda i: (0, 0)),
            pl.BlockSpec((1, H2), lambda i: (0, 0)),
            pl.BlockSpec((1, 1), lambda i: (0, 0)),
        ],
        out_specs=pl.BlockSpec((BM, 1), lambda i: (i, 0)),
        out_shape=jax.ShapeDtypeStruct((B, 1), jnp.float32),
    )


def kernel(U_ids, I_ids, user_table, item_table, W1, b1, W2, b2, Wp, bp):
    B = U_ids.shape[0]
    N, D = user_table.shape
    H1, H2 = W1.shape[1], W2.shape[1]
    utabT = user_table.T
    itabT = item_table.T
    uids = U_ids.astype(jnp.int32)
    iids = I_ids.astype(jnp.int32)
    eu, ei = _make_gather(B, D)(uids, iids, utabT, itabT)
    return _make_mlp(B, D, H1, H2, 2048)(
        eu, ei, W1[:D], W1[D:], b1.reshape(1, H1), W2, b2.reshape(1, H2),
        Wp.reshape(1, H2), bp.reshape(1, 1))


# speedup vs baseline: 1.0040x; 1.0040x over previous
import functools

import jax
import jax.numpy as jnp
from jax import lax
from jax.experimental import pallas as pl
from jax.experimental.pallas import tpu as pltpu
from jax.experimental.pallas import tpu_sc as plsc

_LANES = 16
_CHUNK = 4


@functools.lru_cache(maxsize=None)
def _make_gather(B, D):
    info = plsc.get_sparse_core_info()
    NC, NS = info.num_cores, info.num_subcores
    NW = NC * NS
    b_per_w = B // NW
    n_chunks = b_per_w // _CHUNK
    assert b_per_w * NW == B and n_chunks * _CHUNK == b_per_w

    mesh = plsc.VectorSubcoreMesh(core_axis_name="c", subcore_axis_name="s")

    @functools.partial(
        pl.kernel,
        mesh=mesh,
        out_type=[
            jax.ShapeDtypeStruct((B, D), jnp.float32),
            jax.ShapeDtypeStruct((B, D), jnp.float32),
        ],
        scratch_types=[
            pltpu.VMEM((b_per_w,), jnp.int32),
            pltpu.VMEM((_CHUNK, D, 128), jnp.float32),
            pltpu.VMEM((_CHUNK, D, 128), jnp.float32),
            pltpu.VMEM((2 * _CHUNK, D), jnp.float32),
            pltpu.SemaphoreType.DMA,
            pltpu.SemaphoreType.DMA,
        ],
        compiler_params=pltpu.CompilerParams(needs_layout_passes=False),
    )
    def gather_kernel(uids, iids, utabT, itabT, uout, iout,
                      ids_v, blk_a, blk_b, rows_v, sem_a, sem_b):
        wid = lax.axis_index("s") * NC + lax.axis_index("c")
        base = wid * b_per_w
        iota = lax.iota(jnp.int32, _LANES)

        for ids_hbm, tabT, out in ((uids, utabT, uout), (iids, itabT, iout)):
            pltpu.sync_copy(ids_hbm.at[pl.ds(base, b_per_w)], ids_v)

            def fire(c, blk, sem):
                idv = ids_v[pl.ds(c * _CHUNK, _LANES)]
                for q in range(_CHUNK):
                    col0 = pl.multiple_of((idv[q] >> 7) << 7, 128)
                    pltpu.async_copy(
                        tabT.at[:, pl.ds(col0, 128)], blk.at[q], sem)

            def drain_select(c, blk, sem, half):
                pltpu.make_async_copy(
                    tabT.at[:, pl.ds(0, 128)], blk, sem).wait()
                idv = ids_v[pl.ds(c * _CHUNK, _LANES)]
                for q in range(_CHUNK):
                    lane = jnp.full((_LANES,), idv[q] & 127, jnp.int32)
                    slab = jnp.full((_LANES,), q, jnp.int32)
                    for k in range(D // _LANES):
                        vals = plsc.load_gather(
                            blk, [slab, iota + k * _LANES, lane])
                        plsc.store_scatter(
                            rows_v,
                            [jnp.full((_LANES,), half * _CHUNK + q,
                                      jnp.int32),
                             iota + k * _LANES],
                            vals)

            def pair_body(p, carry):
                c0 = 2 * p
                fire(c0 + 1, blk_b, sem_b)
                drain_select(c0, blk_a, sem_a, 0)

                @pl.when(c0 + 2 < n_chunks)
                def _():
                    fire(c0 + 2, blk_a, sem_a)

                drain_select(c0 + 1, blk_b, sem_b, 1)
                pltpu.sync_copy(
                    rows_v, out.at[pl.ds(base + c0 * _CHUNK, 2 * _CHUNK)])
                return carry

            fire(0, blk_a, sem_a)
            lax.fori_loop(0, n_chunks // 2, pair_body, 0)

    return gather_kernel


def _mlp_body(eu, ei, w1u, w1i, b1, w2, b2, wp, bp, out):
    x = jnp.dot(eu[...], w1u[...], preferred_element_type=jnp.float32)
    x = x + jnp.dot(ei[...], w1i[...], preferred_element_type=jnp.float32)
    h = jnp.maximum(x + b1[...], 0.0)
    h = jnp.maximum(
        jnp.dot(h, w2[...], preferred_element_type=jnp.float32) + b2[...], 0.0)
    out[...] = jnp.sum(h * wp[...], axis=1, keepdims=True) + bp[...]


@functools.lru_cache(maxsize=None)
def _make_mlp(B, D, H1, H2, BM):
    grid = (B // BM,)
    return pl.pallas_call(
        _mlp_body,
        grid=grid,
        in_specs=[
            pl.BlockSpec((BM, D), lambda i: (i, 0)),
            pl.BlockSpec((BM, D), lambda i: (i, 0)),
            pl.BlockSpec((D, H1), lambda i: (0, 0)),
            pl.BlockSpec((D, H1), lambda i: (0, 0)),
            pl.BlockSpec((1, H1), lambda i: (0, 0)),
            pl.BlockSpec((H1, H2), lambda i: (0, 0)),
            pl.BlockSpec((1, H2), lambda i: (0, 0)),
            pl.BlockSpec((1, H2), lambda i: (0, 0)),
            pl.BlockSpec((1, 1), lambda i: (0, 0)),
        ],
        out_specs=pl.BlockSpec((BM, 1), lambda i: (i, 0)),
        out_shape=jax.ShapeDtypeStruct((B, 1), jnp.float32),
    )


def kernel(U_ids, I_ids, user_table, item_table, W1, b1, W2, b2, Wp, bp):
    B = U_ids.shape[0]
    N, D = user_table.shape
    H1, H2 = W1.shape[1], W2.shape[1]
    utabT = user_table.T
    itabT = item_table.T
    uids = U_ids.astype(jnp.int32)
    iids = I_ids.astype(jnp.int32)
    eu, ei = _make_gather(B, D)(uids, iids, utabT, itabT)
    return _make_mlp(B, D, H1, H2, 2048)(
        eu, ei, W1[:D], W1[D:], b1.reshape(1, H1), W2, b2.reshape(1, H2),
        Wp.reshape(1, H2), bp.reshape(1, 1))
